# Initial kernel scaffold; baseline (speedup 1.0000x reference)
import jax
import jax.numpy as jnp
from jax.experimental import pallas as pl


def kernel(x):
    raise NotImplementedError("write your pallas kernel here")


# speedup vs baseline: 1.4733x; 1.4733x over previous
import jax
import jax.numpy as jnp
from jax.experimental import pallas as pl
from jax.experimental.pallas import tpu as pltpu

SB = 256
FB = 512


def _scan_body(x_ref, o_ref, carry_ref):
    s = pl.program_id(2)

    @pl.when(s == 0)
    def _():
        carry_ref[...] = jnp.zeros_like(carry_ref)

    xb = x_ref[0]
    row = jax.lax.broadcasted_iota(jnp.int32, (SB, SB), 0)
    col = jax.lax.broadcasted_iota(jnp.int32, (SB, SB), 1)
    tri = (row >= col).astype(jnp.float32)
    part = jnp.dot(tri, xb, preferred_element_type=jnp.float32)
    y = part + carry_ref[...]
    o_ref[0] = y
    carry_ref[...] = y[SB - 1 : SB, :]


def kernel(x):
    B, S, F = x.shape
    grid = (B, F // FB, S // SB)
    return pl.pallas_call(
        _scan_body,
        grid=grid,
        in_specs=[
            pl.BlockSpec((1, SB, FB), lambda b, f, s: (b, s, f)),
        ],
        out_specs=pl.BlockSpec((1, SB, FB), lambda b, f, s: (b, s, f)),
        out_shape=jax.ShapeDtypeStruct((B, S, F), jnp.float32),
        scratch_shapes=[pltpu.VMEM((1, FB), jnp.float32)],
        compiler_params=pltpu.CompilerParams(
            dimension_semantics=("parallel", "parallel", "arbitrary"),
        ),
    )(x)


# speedup vs baseline: 3.1376x; 2.1296x over previous
import jax
import jax.numpy as jnp
from jax.experimental import pallas as pl
from jax.experimental.pallas import tpu as pltpu

SB = 128
FB = 4096


def _scan_body(x_ref, o_ref, carry_ref):
    s = pl.program_id(2)

    @pl.when(s == 0)
    def _():
        carry_ref[...] = jnp.zeros_like(carry_ref)

    xb = x_ref[0]
    row = jax.lax.broadcasted_iota(jnp.int32, (SB, SB), 0)
    col = jax.lax.broadcasted_iota(jnp.int32, (SB, SB), 1)
    tri = (row >= col).astype(jnp.float32)
    part = jnp.dot(tri, xb, preferred_element_type=jnp.float32)
    y = part + carry_ref[...]
    o_ref[0] = y
    carry_ref[...] = y[SB - 1 : SB, :]


def kernel(x):
    B, S, F = x.shape
    grid = (B, F // FB, S // SB)
    return pl.pallas_call(
        _scan_body,
        grid=grid,
        in_specs=[
            pl.BlockSpec((1, SB, FB), lambda b, f, s: (b, s, f)),
        ],
        out_specs=pl.BlockSpec((1, SB, FB), lambda b, f, s: (b, s, f)),
        out_shape=jax.ShapeDtypeStruct((B, S, F), jnp.float32),
        scratch_shapes=[pltpu.VMEM((1, FB), jnp.float32)],
        compiler_params=pltpu.CompilerParams(
            dimension_semantics=("parallel", "parallel", "arbitrary"),
        ),
    )(x)


# speedup vs baseline: 3.5948x; 1.1457x over previous
import jax
import jax.numpy as jnp
from jax.experimental import pallas as pl
from jax.experimental.pallas import tpu as pltpu

SB = 256
FB = 4096


def _scan_body(x_ref, o_ref, carry_ref):
    s = pl.program_id(2)

    @pl.when(s == 0)
    def _():
        carry_ref[...] = jnp.zeros_like(carry_ref)

    xb = x_ref[0]
    row = jax.lax.broadcasted_iota(jnp.int32, (SB, SB), 0)
    col = jax.lax.broadcasted_iota(jnp.int32, (SB, SB), 1)
    tri = (row >= col).astype(jnp.float32)
    part = jnp.dot(tri, xb, preferred_element_type=jnp.float32)
    y = part + carry_ref[...]
    o_ref[0] = y
    carry_ref[...] = y[SB - 1 : SB, :]


def kernel(x):
    B, S, F = x.shape
    grid = (B, F // FB, S // SB)
    return pl.pallas_call(
        _scan_body,
        grid=grid,
        in_specs=[
            pl.BlockSpec((1, SB, FB), lambda b, f, s: (b, s, f)),
        ],
        out_specs=pl.BlockSpec((1, SB, FB), lambda b, f, s: (b, s, f)),
        out_shape=jax.ShapeDtypeStruct((B, S, F), jnp.float32),
        scratch_shapes=[pltpu.VMEM((1, FB), jnp.float32)],
        compiler_params=pltpu.CompilerParams(
            dimension_semantics=("parallel", "parallel", "arbitrary"),
        ),
    )(x)
